# baseline (speedup 1.0000x reference)
import functools

import jax
import jax.numpy as jnp
from jax import lax
from jax.experimental import pallas as pl
from jax.experimental.pallas import tpu as pltpu
from jax.experimental.pallas import tpu_sc as plsc

N_NODES = 10000
NPAD = 10240
D_IN = 128
D_HID = 128
D_OUT = 64
N_EDGES = 320000
ETOT = N_EDGES + N_NODES
NW = 32
CHUNK = 128
CPW = 84
EPW = CPW * CHUNK
EPAD = EPW * NW
NT = 16
RPT = NPAD // NT

_f32 = jnp.float32
_i32 = jnp.int32


def _make_edge_pass(D, mult_width, spmem_table):
    mesh = plsc.VectorSubcoreMesh(core_axis_name="c", subcore_axis_name="s",
                                  num_cores=2, num_subcores=NT)

    scratch = [
        pltpu.VMEM_SHARED((NPAD, D), _f32),
        pltpu.VMEM_SHARED((NPAD,), _f32),
        pltpu.VMEM_SHARED((NPAD,), _f32),
        pltpu.VMEM_SHARED((NPAD,), _f32),
        [pltpu.VMEM((2, CHUNK), _i32)] * 4,
        [pltpu.VMEM((1, CHUNK), _i32)] * 2,
        [pltpu.VMEM((CHUNK,), _f32)] * 2,
        [pltpu.VMEM((CHUNK, D), _f32)] * 2,
        [pltpu.VMEM((CHUNK,), _f32)] * 2,
        [pltpu.VMEM((CHUNK,), _f32)] * 2,
        pltpu.VMEM((RPT,), _f32),
        [pltpu.SemaphoreType.DMA] * 4,
        [pltpu.SemaphoreType.DMA] * 2,
        [pltpu.SemaphoreType.DMA] * 2,
        [pltpu.SemaphoreType.DMA] * 2,
        [pltpu.SemaphoreType.DMA] * 2,
    ]
    if spmem_table:
        scratch.append(pltpu.VMEM_SHARED((NPAD, D), _f32))

    @functools.partial(
        pl.kernel,
        out_type=(jax.ShapeDtypeStruct((2, NPAD, D), _f32),
                  jax.ShapeDtypeStruct((2, NPAD), _f32)),
        mesh=mesh,
        compiler_params=pltpu.CompilerParams(needs_layout_passes=False),
        scratch_types=scratch,
    )
    def edge_pass(ei_hbm, asrc_hbm, adst_hbm, h_hbm,
                  acc_out, den_out,
                  acc_sh, den_sh, asrc_sh, adst_sh,
                  idx, dstS, wv, rows, asg, adg, zvec,
                  isem, asem, gsem, ssem, wsem, *maybe_h_sh):
        c = lax.axis_index("c")
        s = lax.axis_index("s")
        wid = c * NT + s
        base = s * RPT
        tbl = maybe_h_sh[0] if spmem_table else h_hbm

        def fetch_idx(t, b):
            off = jnp.minimum(wid * EPW + t * CHUNK, N_EDGES - CHUNK)
            pltpu.async_copy(ei_hbm.at[0].at[pl.ds(off, CHUNK)],
                             idx[b].at[0], isem[b])
            pltpu.async_copy(ei_hbm.at[1].at[pl.ds(off, CHUNK)],
                             idx[b].at[1], isem[b])

        def wait_idx(t, b):
            off = jnp.minimum(wid * EPW + t * CHUNK, N_EDGES - CHUNK)
            pltpu.make_async_copy(ei_hbm.at[0].at[pl.ds(off, CHUNK)],
                                  idx[b].at[0], isem[b]).wait()
            pltpu.make_async_copy(ei_hbm.at[1].at[pl.ds(off, CHUNK)],
                                  idx[b].at[1], isem[b]).wait()
            g = wid * EPW + t * CHUNK

            @pl.when(g >= N_EDGES)
            def _():
                for j in range(CHUNK // 16):
                    node = (g - N_EDGES + j * 16) + lax.iota(_i32, 16)
                    real = node < N_NODES
                    idx[b][0, pl.ds(j * 16, 16)] = jnp.where(
                        real, node, node & 63)
                    idx[b][1, pl.ds(j * 16, 16)] = jnp.where(
                        real, node, N_NODES + (node & 127))

        def fetch_a(bi, ba):
            pltpu.async_copy(asrc_sh.at[idx[bi].at[0]], asg[ba], asem[ba])
            pltpu.async_copy(adst_sh.at[idx[bi].at[1]], adg[ba], asem[ba])

        def wait_a(bi, ba):
            pltpu.make_async_copy(asrc_sh.at[idx[bi].at[0]], asg[ba],
                                  asem[ba]).wait()
            pltpu.make_async_copy(adst_sh.at[idx[bi].at[1]], adg[ba],
                                  asem[ba]).wait()

        def zrow(r, carry):
            for k in range(D // 16):
                rows[0][r, pl.ds(k * 16, 16)] = jnp.zeros((16,), _f32)
            return carry
        lax.fori_loop(0, CHUNK, zrow, 0)

        def zv(m, carry):
            zvec[pl.ds(m * 16, 16)] = jnp.zeros((16,), _f32)
            return carry
        lax.fori_loop(0, RPT // 16, zv, 0)

        for t in range(RPT // CHUNK):
            pltpu.sync_copy(rows[0], acc_sh.at[pl.ds(base + t * CHUNK, CHUNK)])
        pltpu.sync_copy(zvec, den_sh.at[pl.ds(base, RPT)])

        @pl.when(s == 0)
        def _():
            pltpu.sync_copy(asrc_hbm, asrc_sh)

        @pl.when(s == 1)
        def _():
            pltpu.sync_copy(adst_hbm, adst_sh)

        if spmem_table:
            pltpu.sync_copy(h_hbm.at[pl.ds(base, RPT)],
                            tbl.at[pl.ds(base, RPT)])

        fetch_idx(0, 0)
        fetch_idx(1, 1)
        plsc.subcore_barrier()
        wait_idx(0, 0)
        fetch_a(0, 0)

        def do_multiply_scatter(b1):
            @plsc.parallel_loop(0, CHUNK, step=1, unroll=4)
            def _(r):
                wb = plsc.load_gather(wv[b1], [jnp.broadcast_to(r, (16,))])
                for k in range(mult_width // 16):
                    rows[b1][r, pl.ds(k * 16, 16)] = (
                        rows[b1][r, pl.ds(k * 16, 16)] * wb)

            pltpu.async_copy(rows[b1], acc_sh.at[dstS[b1].at[0]], ssem[b1],
                             add=True)

        def body(t, bi, b):
            b1 = 1 - b
            bi1 = (bi + 1) % 4
            bi2 = (bi + 2) % 4

            @pl.when(t >= 2)
            def _():
                pltpu.make_async_copy(wv[b], den_sh.at[idx[bi2].at[1]],
                                      wsem[b]).wait()

            @pl.when(t + 2 < CPW)
            def _():
                fetch_idx(t + 2, bi2)

            @pl.when(t + 1 < CPW)
            def _():
                wait_idx(t + 1, bi1)
                fetch_a(bi1, b1)

            wait_a(bi, b)

            @pl.when(t >= 2)
            def _():
                pltpu.make_async_copy(rows[b], acc_sh.at[dstS[b].at[0]],
                                      ssem[b]).wait()
            for j in range(CHUNK // 16):
                e = asg[b][pl.ds(j * 16, 16)] + adg[b][pl.ds(j * 16, 16)]
                e = jnp.where(e >= 0.0, e, e * 0.2)
                wv[b][pl.ds(j * 16, 16)] = jnp.exp(e)
                dstS[b][0, pl.ds(j * 16, 16)] = idx[bi][1, pl.ds(j * 16, 16)]
            pltpu.async_copy(wv[b], den_sh.at[idx[bi].at[1]], wsem[b],
                             add=True)
            pltpu.async_copy(tbl.at[idx[bi].at[0]], rows[b], gsem[b])

            @pl.when(t >= 1)
            def _():
                pltpu.make_async_copy(tbl.at[idx[(bi - 1) % 4].at[0]],
                                      rows[b1], gsem[b1]).wait()
                do_multiply_scatter(b1)

        def gbody(g, carry):
            for q in range(4):
                body(4 * g + q, q, q % 2)
            return carry
        lax.fori_loop(0, CPW // 4, gbody, 0)

        bL = (CPW - 1) % 2
        pltpu.make_async_copy(tbl.at[idx[(CPW - 1) % 4].at[0]], rows[bL],
                              gsem[bL]).wait()
        do_multiply_scatter(bL)
        pltpu.make_async_copy(wv[0], den_sh.at[idx[0].at[1]], wsem[0]).wait()
        pltpu.make_async_copy(wv[1], den_sh.at[idx[1].at[1]], wsem[1]).wait()
        pltpu.make_async_copy(rows[0], acc_sh.at[dstS[0].at[0]],
                              ssem[0]).wait()
        pltpu.make_async_copy(rows[1], acc_sh.at[dstS[1].at[0]],
                              ssem[1]).wait()

        plsc.subcore_barrier()
        pltpu.sync_copy(acc_sh.at[pl.ds(base, RPT)],
                        acc_out.at[c].at[pl.ds(base, RPT)])
        pltpu.sync_copy(den_sh.at[pl.ds(base, RPT)],
                        den_out.at[c].at[pl.ds(base, RPT)])

    return edge_pass


_edge_pass1 = _make_edge_pass(D_HID, D_HID, False)
_edge_pass2 = _make_edge_pass(D_HID, D_OUT, False)


_BLK = 1024
_GRID = NPAD // _BLK


def _dense1_body(x_ref, w_ref, asc_ref, adc_ref, h_ref, as_ref, ad_ref):
    h = jnp.dot(x_ref[...], w_ref[...], preferred_element_type=_f32)
    h_ref[...] = h
    as_ref[...] = jnp.sum(h * asc_ref[...], axis=1)
    ad_ref[...] = jnp.sum(h * adc_ref[...], axis=1)


def _dense1(x_pad, W1, asc, adc):
    return pl.pallas_call(
        _dense1_body,
        grid=(_GRID,),
        in_specs=[
            pl.BlockSpec((_BLK, D_IN), lambda i: (i, 0)),
            pl.BlockSpec((D_IN, D_HID), lambda i: (0, 0)),
            pl.BlockSpec((1, D_HID), lambda i: (0, 0)),
            pl.BlockSpec((1, D_HID), lambda i: (0, 0)),
        ],
        out_specs=[
            pl.BlockSpec((_BLK, D_HID), lambda i: (i, 0)),
            pl.BlockSpec((_BLK,), lambda i: (i,)),
            pl.BlockSpec((_BLK,), lambda i: (i,)),
        ],
        out_shape=[
            jax.ShapeDtypeStruct((NPAD, D_HID), _f32),
            jax.ShapeDtypeStruct((NPAD,), _f32),
            jax.ShapeDtypeStruct((NPAD,), _f32),
        ],
    )(x_pad, W1, asc, adc)


def _dense2_body(acc_ref, den_ref, b_ref, w_ref, asc_ref, adc_ref,
                 h_ref, as_ref, ad_ref):
    den = den_ref[0] + den_ref[1]
    x2 = (acc_ref[0] + acc_ref[1]) / (den[:, None] + 1e-16) + b_ref[...]
    x2 = jnp.maximum(x2, 0.0)
    h2 = jnp.dot(x2, w_ref[...], preferred_element_type=_f32)
    h_ref[...] = h2
    as_ref[...] = jnp.sum(h2 * asc_ref[...], axis=1)
    ad_ref[...] = jnp.sum(h2 * adc_ref[...], axis=1)


def _dense2(acc, den, b1, W2, asc, adc):
    return pl.pallas_call(
        _dense2_body,
        grid=(_GRID,),
        in_specs=[
            pl.BlockSpec((2, _BLK, D_HID), lambda i: (0, i, 0)),
            pl.BlockSpec((2, _BLK), lambda i: (0, i)),
            pl.BlockSpec((1, D_HID), lambda i: (0, 0)),
            pl.BlockSpec((D_HID, D_HID), lambda i: (0, 0)),
            pl.BlockSpec((1, D_HID), lambda i: (0, 0)),
            pl.BlockSpec((1, D_HID), lambda i: (0, 0)),
        ],
        out_specs=[
            pl.BlockSpec((_BLK, D_HID), lambda i: (i, 0)),
            pl.BlockSpec((_BLK,), lambda i: (i,)),
            pl.BlockSpec((_BLK,), lambda i: (i,)),
        ],
        out_shape=[
            jax.ShapeDtypeStruct((NPAD, D_HID), _f32),
            jax.ShapeDtypeStruct((NPAD,), _f32),
            jax.ShapeDtypeStruct((NPAD,), _f32),
        ],
    )(acc, den, b1, W2, asc, adc)


def _final_body(acc_ref, den_ref, b_ref, out_ref):
    den = den_ref[0] + den_ref[1]
    o = (acc_ref[0, :, :D_OUT] + acc_ref[1, :, :D_OUT]) / (den[:, None] + 1e-16)
    o = o + b_ref[...]
    m = jnp.max(o, axis=1, keepdims=True)
    lse = jnp.log(jnp.sum(jnp.exp(o - m), axis=1, keepdims=True)) + m
    out_ref[...] = o - lse


def _final(acc, den, b2):
    return pl.pallas_call(
        _final_body,
        grid=(_GRID,),
        in_specs=[
            pl.BlockSpec((2, _BLK, D_HID), lambda i: (0, i, 0)),
            pl.BlockSpec((2, _BLK), lambda i: (0, i)),
            pl.BlockSpec((1, D_OUT), lambda i: (0, 0)),
        ],
        out_specs=pl.BlockSpec((_BLK, D_OUT), lambda i: (i, 0)),
        out_shape=jax.ShapeDtypeStruct((N_NODES, D_OUT), _f32),
    )(acc, den, b2)


def kernel(x, edge_index, W1, att_src1, att_dst1, bias1,
           W2, att_src2, att_dst2, bias2):
    ei = edge_index.astype(_i32)

    W2p = jnp.pad(W2, ((0, 0), (0, D_HID - D_OUT)))
    as2p = jnp.pad(att_src2, (0, D_HID - D_OUT)).reshape(1, D_HID)
    ad2p = jnp.pad(att_dst2, (0, D_HID - D_OUT)).reshape(1, D_HID)

    h1, as1, ad1 = _dense1(x, W1,
                           att_src1.reshape(1, D_HID),
                           att_dst1.reshape(1, D_HID))
    acc1, den1 = _edge_pass1(ei, as1, ad1, h1)
    h2, as2, ad2 = _dense2(acc1, den1, bias1.reshape(1, D_HID), W2p,
                           as2p, ad2p)
    acc2, den2 = _edge_pass2(ei, as2, ad2, h2)
    return _final(acc2, den2, bias2.reshape(1, D_OUT))


# speedup vs baseline: 1.0027x; 1.0027x over previous
import functools

import jax
import jax.numpy as jnp
from jax import lax
from jax.experimental import pallas as pl
from jax.experimental.pallas import tpu as pltpu
from jax.experimental.pallas import tpu_sc as plsc

N_NODES = 10000
NPAD = 10240
D_IN = 128
D_HID = 128
D_OUT = 64
N_EDGES = 320000
ETOT = N_EDGES + N_NODES
NW = 32
CHUNK = 128
CPW = 84
EPW = CPW * CHUNK
EPAD = EPW * NW
NT = 16
RPT = NPAD // NT

_f32 = jnp.float32
_i32 = jnp.int32


def _make_edge_pass(D, mult_width):
    D_acc = D
    compact = False
    mesh = plsc.VectorSubcoreMesh(core_axis_name="c", subcore_axis_name="s",
                                  num_cores=2, num_subcores=NT)

    scratch = [
        pltpu.VMEM_SHARED((NPAD, D_acc), _f32),
        pltpu.VMEM_SHARED((NPAD,), _f32),
        pltpu.VMEM_SHARED((NPAD,), _f32),
        pltpu.VMEM_SHARED((NPAD,), _f32),
        [pltpu.VMEM((2, CHUNK), _i32)] * 4,
        [pltpu.VMEM((1, CHUNK), _i32)] * 2,
        [pltpu.VMEM((CHUNK,), _f32)] * 2,
        [pltpu.VMEM((CHUNK, D), _f32)] * 2,
        [pltpu.VMEM((CHUNK,), _f32)] * 2,
        [pltpu.VMEM((CHUNK,), _f32)] * 2,
        pltpu.VMEM((RPT,), _f32),
        [pltpu.SemaphoreType.DMA] * 4,
        [pltpu.SemaphoreType.DMA] * 2,
        [pltpu.SemaphoreType.DMA] * 2,
        [pltpu.SemaphoreType.DMA] * 2,
        [pltpu.SemaphoreType.DMA] * 2,
    ]
    if compact:
        scratch.append([pltpu.VMEM((CHUNK, D_acc), _f32)] * 2)

    @functools.partial(
        pl.kernel,
        out_type=(jax.ShapeDtypeStruct((2, NPAD, D_acc), _f32),
                  jax.ShapeDtypeStruct((2, NPAD), _f32)),
        mesh=mesh,
        compiler_params=pltpu.CompilerParams(needs_layout_passes=False),
        scratch_types=scratch,
    )
    def edge_pass(ei_hbm, asrc_hbm, adst_hbm, h_hbm,
                  acc_out, den_out,
                  acc_sh, den_sh, asrc_sh, adst_sh,
                  idx, dstS, wv, rows, asg, adg, zvec,
                  isem, asem, gsem, ssem, wsem, *maybe_rows_c):
        c = lax.axis_index("c")
        s = lax.axis_index("s")
        wid = c * NT + s
        base = s * RPT
        out_rows = maybe_rows_c[0] if compact else rows

        def fetch_idx(t, b):
            off = jnp.minimum(wid * EPW + t * CHUNK, N_EDGES - CHUNK)
            pltpu.async_copy(ei_hbm.at[0].at[pl.ds(off, CHUNK)],
                             idx[b].at[0], isem[b])
            pltpu.async_copy(ei_hbm.at[1].at[pl.ds(off, CHUNK)],
                             idx[b].at[1], isem[b])

        def wait_idx(t, b):
            off = jnp.minimum(wid * EPW + t * CHUNK, N_EDGES - CHUNK)
            pltpu.make_async_copy(ei_hbm.at[0].at[pl.ds(off, CHUNK)],
                                  idx[b].at[0], isem[b]).wait()
            pltpu.make_async_copy(ei_hbm.at[1].at[pl.ds(off, CHUNK)],
                                  idx[b].at[1], isem[b]).wait()
            g = wid * EPW + t * CHUNK

            @pl.when(g >= N_EDGES)
            def _():
                for j in range(CHUNK // 16):
                    node = (g - N_EDGES + j * 16) + lax.iota(_i32, 16)
                    real = node < N_NODES
                    idx[b][0, pl.ds(j * 16, 16)] = jnp.where(
                        real, node, node & 63)
                    idx[b][1, pl.ds(j * 16, 16)] = jnp.where(
                        real, node, N_NODES + (node & 127))

        def fetch_a(bi, ba):
            pltpu.async_copy(asrc_sh.at[idx[bi].at[0]], asg[ba], asem[ba])
            pltpu.async_copy(adst_sh.at[idx[bi].at[1]], adg[ba], asem[ba])

        def wait_a(bi, ba):
            pltpu.make_async_copy(asrc_sh.at[idx[bi].at[0]], asg[ba],
                                  asem[ba]).wait()
            pltpu.make_async_copy(adst_sh.at[idx[bi].at[1]], adg[ba],
                                  asem[ba]).wait()

        def zrow(r, carry):
            for k in range(D_acc // 16):
                out_rows[0][r, pl.ds(k * 16, 16)] = jnp.zeros((16,), _f32)
            return carry
        lax.fori_loop(0, CHUNK, zrow, 0)

        def zv(m, carry):
            zvec[pl.ds(m * 16, 16)] = jnp.zeros((16,), _f32)
            return carry
        lax.fori_loop(0, RPT // 16, zv, 0)

        for t in range(RPT // CHUNK):
            pltpu.sync_copy(out_rows[0],
                            acc_sh.at[pl.ds(base + t * CHUNK, CHUNK)])
        pltpu.sync_copy(zvec, den_sh.at[pl.ds(base, RPT)])

        @pl.when(s == 0)
        def _():
            pltpu.sync_copy(asrc_hbm, asrc_sh)

        @pl.when(s == 1)
        def _():
            pltpu.sync_copy(adst_hbm, adst_sh)

        fetch_idx(0, 0)
        fetch_idx(1, 1)
        plsc.subcore_barrier()
        wait_idx(0, 0)
        fetch_a(0, 0)

        def do_multiply_scatter(b1):
            @plsc.parallel_loop(0, CHUNK, step=1, unroll=4)
            def _(r):
                wb = plsc.load_gather(wv[b1], [jnp.broadcast_to(r, (16,))])
                for k in range(mult_width // 16):
                    out_rows[b1][r, pl.ds(k * 16, 16)] = (
                        rows[b1][r, pl.ds(k * 16, 16)] * wb)

            pltpu.async_copy(out_rows[b1], acc_sh.at[dstS[b1].at[0]], ssem[b1],
                             add=True)

        def body(t, bi, b):
            b1 = 1 - b
            bi1 = (bi + 1) % 4
            bi2 = (bi + 2) % 4

            @pl.when(t >= 2)
            def _():
                pltpu.make_async_copy(wv[b], den_sh.at[idx[bi2].at[1]],
                                      wsem[b]).wait()

            @pl.when(t + 2 < CPW)
            def _():
                fetch_idx(t + 2, bi2)

            @pl.when(t + 1 < CPW)
            def _():
                wait_idx(t + 1, bi1)
                fetch_a(bi1, b1)

            wait_a(bi, b)

            @pl.when(t >= 2)
            def _():
                pltpu.make_async_copy(out_rows[b], acc_sh.at[dstS[b].at[0]],
                                      ssem[b]).wait()
            for j in range(CHUNK // 16):
                e = asg[b][pl.ds(j * 16, 16)] + adg[b][pl.ds(j * 16, 16)]
                e = jnp.where(e >= 0.0, e, e * 0.2)
                wv[b][pl.ds(j * 16, 16)] = jnp.exp(e)
                dstS[b][0, pl.ds(j * 16, 16)] = idx[bi][1, pl.ds(j * 16, 16)]
            pltpu.async_copy(wv[b], den_sh.at[idx[bi].at[1]], wsem[b],
                             add=True)
            pltpu.async_copy(h_hbm.at[idx[bi].at[0]], rows[b], gsem[b])

            @pl.when(t >= 1)
            def _():
                pltpu.make_async_copy(h_hbm.at[idx[(bi - 1) % 4].at[0]],
                                      rows[b1], gsem[b1]).wait()
                do_multiply_scatter(b1)

        def gbody(g, carry):
            for q in range(4):
                body(4 * g + q, q, q % 2)
            return carry
        lax.fori_loop(0, CPW // 4, gbody, 0)

        bL = (CPW - 1) % 2
        pltpu.make_async_copy(h_hbm.at[idx[(CPW - 1) % 4].at[0]], rows[bL],
                              gsem[bL]).wait()
        do_multiply_scatter(bL)
        pltpu.make_async_copy(wv[0], den_sh.at[idx[0].at[1]], wsem[0]).wait()
        pltpu.make_async_copy(wv[1], den_sh.at[idx[1].at[1]], wsem[1]).wait()
        pltpu.make_async_copy(out_rows[0], acc_sh.at[dstS[0].at[0]],
                              ssem[0]).wait()
        pltpu.make_async_copy(out_rows[1], acc_sh.at[dstS[1].at[0]],
                              ssem[1]).wait()

        plsc.subcore_barrier()
        pltpu.sync_copy(acc_sh.at[pl.ds(base, RPT)],
                        acc_out.at[c].at[pl.ds(base, RPT)])
        pltpu.sync_copy(den_sh.at[pl.ds(base, RPT)],
                        den_out.at[c].at[pl.ds(base, RPT)])

    return edge_pass


_edge_pass1 = _make_edge_pass(D_HID, D_HID)
_edge_pass2 = _make_edge_pass(D_HID, D_OUT)


_BLK = 1024
_GRID = NPAD // _BLK


def _dense1_body(x_ref, w_ref, asc_ref, adc_ref, h_ref, as_ref, ad_ref):
    h = jnp.dot(x_ref[...], w_ref[...], preferred_element_type=_f32)
    h_ref[...] = h
    as_ref[...] = jnp.sum(h * asc_ref[...], axis=1)
    ad_ref[...] = jnp.sum(h * adc_ref[...], axis=1)


def _dense1(x_pad, W1, asc, adc):
    return pl.pallas_call(
        _dense1_body,
        grid=(_GRID,),
        in_specs=[
            pl.BlockSpec((_BLK, D_IN), lambda i: (i, 0)),
            pl.BlockSpec((D_IN, D_HID), lambda i: (0, 0)),
            pl.BlockSpec((1, D_HID), lambda i: (0, 0)),
            pl.BlockSpec((1, D_HID), lambda i: (0, 0)),
        ],
        out_specs=[
            pl.BlockSpec((_BLK, D_HID), lambda i: (i, 0)),
            pl.BlockSpec((_BLK,), lambda i: (i,)),
            pl.BlockSpec((_BLK,), lambda i: (i,)),
        ],
        out_shape=[
            jax.ShapeDtypeStruct((NPAD, D_HID), _f32),
            jax.ShapeDtypeStruct((NPAD,), _f32),
            jax.ShapeDtypeStruct((NPAD,), _f32),
        ],
    )(x_pad, W1, asc, adc)


def _dense2_body(acc_ref, den_ref, b_ref, w_ref, asc_ref, adc_ref,
                 h_ref, as_ref, ad_ref):
    den = den_ref[0] + den_ref[1]
    x2 = (acc_ref[0] + acc_ref[1]) / (den[:, None] + 1e-16) + b_ref[...]
    x2 = jnp.maximum(x2, 0.0)
    h2 = jnp.dot(x2, w_ref[...], preferred_element_type=_f32)
    h_ref[...] = h2
    as_ref[...] = jnp.sum(h2 * asc_ref[...], axis=1)
    ad_ref[...] = jnp.sum(h2 * adc_ref[...], axis=1)


def _dense2(acc, den, b1, W2, asc, adc):
    return pl.pallas_call(
        _dense2_body,
        grid=(_GRID,),
        in_specs=[
            pl.BlockSpec((2, _BLK, D_HID), lambda i: (0, i, 0)),
            pl.BlockSpec((2, _BLK), lambda i: (0, i)),
            pl.BlockSpec((1, D_HID), lambda i: (0, 0)),
            pl.BlockSpec((D_HID, D_HID), lambda i: (0, 0)),
            pl.BlockSpec((1, D_HID), lambda i: (0, 0)),
            pl.BlockSpec((1, D_HID), lambda i: (0, 0)),
        ],
        out_specs=[
            pl.BlockSpec((_BLK, D_HID), lambda i: (i, 0)),
            pl.BlockSpec((_BLK,), lambda i: (i,)),
            pl.BlockSpec((_BLK,), lambda i: (i,)),
        ],
        out_shape=[
            jax.ShapeDtypeStruct((NPAD, D_HID), _f32),
            jax.ShapeDtypeStruct((NPAD,), _f32),
            jax.ShapeDtypeStruct((NPAD,), _f32),
        ],
    )(acc, den, b1, W2, asc, adc)


def _final_body(acc_ref, den_ref, b_ref, out_ref):
    den = den_ref[0] + den_ref[1]
    o = (acc_ref[0, :, :D_OUT] + acc_ref[1, :, :D_OUT]) / (den[:, None] + 1e-16)
    o = o + b_ref[...]
    m = jnp.max(o, axis=1, keepdims=True)
    lse = jnp.log(jnp.sum(jnp.exp(o - m), axis=1, keepdims=True)) + m
    out_ref[...] = o - lse


def _final(acc, den, b2):
    return pl.pallas_call(
        _final_body,
        grid=(_GRID,),
        in_specs=[
            pl.BlockSpec((2, _BLK, D_HID), lambda i: (0, i, 0)),
            pl.BlockSpec((2, _BLK), lambda i: (0, i)),
            pl.BlockSpec((1, D_OUT), lambda i: (0, 0)),
        ],
        out_specs=pl.BlockSpec((_BLK, D_OUT), lambda i: (i, 0)),
        out_shape=jax.ShapeDtypeStruct((N_NODES, D_OUT), _f32),
    )(acc, den, b2)


def kernel(x, edge_index, W1, att_src1, att_dst1, bias1,
           W2, att_src2, att_dst2, bias2):
    ei = edge_index.astype(_i32)

    W2p = jnp.pad(W2, ((0, 0), (0, D_HID - D_OUT)))
    as2p = jnp.pad(att_src2, (0, D_HID - D_OUT)).reshape(1, D_HID)
    ad2p = jnp.pad(att_dst2, (0, D_HID - D_OUT)).reshape(1, D_HID)

    h1, as1, ad1 = _dense1(x, W1,
                           att_src1.reshape(1, D_HID),
                           att_dst1.reshape(1, D_HID))
    acc1, den1 = _edge_pass1(ei, as1, ad1, h1)
    h2, as2, ad2 = _dense2(acc1, den1, bias1.reshape(1, D_HID), W2p,
                           as2p, ad2p)
    acc2, den2 = _edge_pass2(ei, as2, ad2, h2)
    return _final(acc2, den2, bias2.reshape(1, D_OUT))
